# baseline (speedup 1.0000x reference)
import functools

import jax
import jax.numpy as jnp
from jax import lax
from jax.experimental import pallas as pl
from jax.experimental.pallas import tpu as pltpu
from jax.experimental.pallas import tpu_sc as plsc

DIM = 64
PAD = 128
CB = 1024
LANE_BLOCK = 1024
BATCH_BLOCK = 8


def _argmin_body(xt_ref, et_ref, ind_ref, indb_ref, pad_ref):
    et = et_ref[...]
    etT = et.T
    ee = jnp.sum(etT * etT, axis=1, keepdims=True)
    et2 = et + et
    rpb = LANE_BLOCK // PAD
    for bb in range(BATCH_BLOCK):
        xb = xt_ref[bb]
        xx = jnp.sum(xb * xb, axis=0)[None, :]
        xe2 = lax.dot_general(et2, xb, (((0,), (0,)), ((), ())),
                              preferred_element_type=jnp.float32)
        d = (xx - xe2) + ee
        m = jnp.min(d, axis=0, keepdims=True)
        iota = lax.broadcasted_iota(jnp.int32, (CB, 1), 0).astype(jnp.float32)
        ind_f = jnp.min(jnp.where(d <= m, iota, jnp.float32(2**30)), axis=0)
        ind_i = ind_f.astype(jnp.int32)
        ind_ref[pl.ds(bb * rpb, rpb), :] = ind_i.reshape(rpb, PAD)
        indb_ref[pl.ds(bb, 1), :] = ind_i.reshape(1, LANE_BLOCK)
    pad_ref[:, :DIM] = etT
    pad_ref[:, DIM:] = jnp.zeros((CB, PAD - DIM), jnp.float32)


def _argmin_indices(xt, et):
    nb = xt.shape[0]
    n = nb * xt.shape[2]
    grid = nb // BATCH_BLOCK
    rows_per_step = BATCH_BLOCK * LANE_BLOCK // PAD
    ind2d, indb, embed_pad = pl.pallas_call(
        _argmin_body,
        grid=(grid,),
        in_specs=[
            pl.BlockSpec((BATCH_BLOCK, DIM, LANE_BLOCK), lambda i: (i, 0, 0)),
            pl.BlockSpec((DIM, CB), lambda i: (0, 0)),
        ],
        out_specs=[
            pl.BlockSpec((rows_per_step, PAD), lambda i: (i, 0)),
            pl.BlockSpec((BATCH_BLOCK, LANE_BLOCK), lambda i: (i, 0)),
            pl.BlockSpec((CB, PAD), lambda i: (0, 0)),
        ],
        out_shape=[
            jax.ShapeDtypeStruct((n // PAD, PAD), jnp.int32),
            jax.ShapeDtypeStruct((nb, LANE_BLOCK), jnp.int32),
            jax.ShapeDtypeStruct((CB, PAD), jnp.float32),
        ],
    )(xt, et)
    return ind2d, indb, embed_pad


@functools.lru_cache(maxsize=None)
def _sc_gather_fn(batch):
    info = plsc.get_sparse_core_info()
    nc = info.num_cores
    nw = nc * info.num_subcores
    nrow = batch // PAD
    rows_per_w = nrow // nw
    mesh = plsc.VectorSubcoreMesh(core_axis_name="c", subcore_axis_name="s")

    @functools.partial(
        pl.kernel,
        mesh=mesh,
        out_type=jax.ShapeDtypeStruct((nrow, PAD, PAD), jnp.float32),
        scratch_types=[
            pltpu.VMEM((rows_per_w, PAD), jnp.int32),
            pltpu.VMEM((rows_per_w, PAD, PAD), jnp.float32),
            pltpu.SemaphoreType.DMA,
            pltpu.SemaphoreType.DMA,
        ],
    )
    def gather(table_hbm, idx_hbm, out_hbm, idx_v, rows_v, gsem, ssem):
        wid = lax.axis_index("s") * nc + lax.axis_index("c")
        base = wid * rows_per_w
        pltpu.sync_copy(idx_hbm.at[pl.ds(base, rows_per_w)], idx_v)
        gathers = [
            pltpu.async_copy(table_hbm.at[idx_v.at[j]], rows_v.at[j], gsem)
            for j in range(rows_per_w)
        ]
        stores = []
        for j in range(rows_per_w):
            gathers[j].wait()
            stores.append(
                pltpu.async_copy(rows_v.at[j], out_hbm.at[base + j], ssem))
        for s in stores:
            s.wait()

    return gather


def kernel(x, embed):
    shape = x.shape
    n = x.shape[0] * x.shape[1]
    x = x.astype(jnp.float32)
    xt = jnp.transpose(x, (0, 2, 1))
    et = jnp.transpose(embed.astype(jnp.float32))
    ind2d, indb, embed_pad = _argmin_indices(xt, et)
    rows = _sc_gather_fn(n)(embed_pad, ind2d)
    quantize = rows.reshape(n, PAD)[:, :DIM]
    return (quantize.reshape(shape).astype(x.dtype),
            indb.reshape(shape[:-1]))


# speedup vs baseline: 1.0116x; 1.0116x over previous
import functools

import jax
import jax.numpy as jnp
from jax import lax
from jax.experimental import pallas as pl
from jax.experimental.pallas import tpu as pltpu
from jax.experimental.pallas import tpu_sc as plsc

DIM = 64
PAD = 128
CB = 1024
LANE_BLOCK = 1024
BATCH_BLOCK = 8


def _argmin_body(xt_ref, et_ref, ind_ref, indb_ref, pad_ref):
    et = et_ref[...]
    etT = et.T
    ee = jnp.sum(etT * etT, axis=1, keepdims=True)
    et2 = et + et
    rpb = LANE_BLOCK // PAD
    for bb in range(BATCH_BLOCK):
        xb = xt_ref[bb]
        xx = jnp.sum(xb * xb, axis=0)[None, :]
        xe2 = lax.dot_general(et2, xb, (((0,), (0,)), ((), ())),
                              preferred_element_type=jnp.float32)
        d = (xx - xe2) + ee
        m = jnp.min(d, axis=0, keepdims=True)
        iota = lax.broadcasted_iota(jnp.int32, (CB, 1), 0).astype(jnp.float32)
        ind_f = jnp.min(jnp.where(d <= m, iota, jnp.float32(2**30)), axis=0)
        ind_i = ind_f.astype(jnp.int32)
        ind_ref[pl.ds(bb * rpb, rpb), :] = ind_i.reshape(rpb, PAD)
        indb_ref[pl.ds(bb, 1), :] = ind_i.reshape(1, LANE_BLOCK)
    pad_ref[:, :DIM] = etT
    pad_ref[:, DIM:] = jnp.zeros((CB, PAD - DIM), jnp.float32)


def _argmin_indices(xt, et):
    nb = xt.shape[0]
    n = nb * xt.shape[2]
    grid = nb // BATCH_BLOCK
    rows_per_step = BATCH_BLOCK * LANE_BLOCK // PAD
    ind2d, indb, embed_pad = pl.pallas_call(
        _argmin_body,
        grid=(grid,),
        in_specs=[
            pl.BlockSpec((BATCH_BLOCK, DIM, LANE_BLOCK), lambda i: (i, 0, 0)),
            pl.BlockSpec((DIM, CB), lambda i: (0, 0)),
        ],
        out_specs=[
            pl.BlockSpec((rows_per_step, PAD), lambda i: (i, 0)),
            pl.BlockSpec((BATCH_BLOCK, LANE_BLOCK), lambda i: (i, 0)),
            pl.BlockSpec((CB, PAD), lambda i: (0, 0)),
        ],
        out_shape=[
            jax.ShapeDtypeStruct((n // PAD, PAD), jnp.int32),
            jax.ShapeDtypeStruct((nb, LANE_BLOCK), jnp.int32),
            jax.ShapeDtypeStruct((CB, PAD), jnp.float32),
        ],
    )(xt, et)
    return ind2d, indb, embed_pad


@functools.lru_cache(maxsize=None)
def _sc_gather_fn(batch):
    info = plsc.get_sparse_core_info()
    nc = info.num_cores
    nw = nc * info.num_subcores
    nrow = batch // PAD
    rows_per_w = nrow // nw
    mesh = plsc.VectorSubcoreMesh(core_axis_name="c", subcore_axis_name="s")

    @functools.partial(
        pl.kernel,
        mesh=mesh,
        out_type=jax.ShapeDtypeStruct((nrow, PAD, PAD), jnp.float32),
        scratch_types=[
            pltpu.VMEM((rows_per_w, PAD), jnp.int32),
            pltpu.VMEM((rows_per_w, PAD, PAD), jnp.float32),
            pltpu.SemaphoreType.DMA,
        ],
    )
    def gather(table_hbm, idx_hbm, out_hbm, idx_v, rows_v, sem):
        wid = lax.axis_index("s") * nc + lax.axis_index("c")
        base = wid * rows_per_w
        pltpu.sync_copy(idx_hbm.at[pl.ds(base, rows_per_w)], idx_v)
        copies = [
            pltpu.async_copy(table_hbm.at[idx_v.at[j]], rows_v.at[j], sem)
            for j in range(rows_per_w)
        ]
        for c in copies:
            c.wait()
        pltpu.sync_copy(rows_v, out_hbm.at[pl.ds(base, rows_per_w)])

    return gather


def kernel(x, embed):
    shape = x.shape
    n = x.shape[0] * x.shape[1]
    x = x.astype(jnp.float32)
    xt = jnp.transpose(x, (0, 2, 1))
    et = jnp.transpose(embed.astype(jnp.float32))
    ind2d, indb, embed_pad = _argmin_indices(xt, et)
    rows = _sc_gather_fn(n)(embed_pad, ind2d)
    quantize = rows.reshape(n, PAD)[:, :DIM]
    return (quantize.reshape(shape).astype(x.dtype),
            indb.reshape(shape[:-1]))
